# Initial kernel scaffold; baseline (speedup 1.0000x reference)
import jax
import jax.numpy as jnp
from jax.experimental import pallas as pl


def kernel(x, edge_index, W0, W1):
    raise NotImplementedError("write your pallas kernel here")


# speedup vs baseline: 5.6384x; 5.6384x over previous
import functools

import jax
import jax.numpy as jnp
from jax import lax
from jax.experimental import pallas as pl
from jax.experimental.pallas import tpu as pltpu
from jax.experimental.pallas import tpu_sc as plsc

N_NODES = 10000
N_EDGES = 320000
D_FEAT = 128
HIDDEN = 100
OUT_DIM = 10

D1 = 112
D2 = 16
NC = 2
NS = 16
CHUNK = 80
EDGES_PER_TILE = N_EDGES // (NC * NS)
ITERS = EDGES_PER_TILE // CHUNK
ROWS_PER_TILE = 624
ROWS_TAIL = N_NODES - NS * ROWS_PER_TILE


def _mm_body(x_ref, w_ref, o_ref):
    o_ref[...] = jnp.dot(x_ref[...], w_ref[...],
                         preferred_element_type=jnp.float32)


def _matmul(x, w):
    m, k = x.shape
    n = w.shape[1]
    blk = 1000
    return pl.pallas_call(
        _mm_body,
        grid=(m // blk,),
        in_specs=[
            pl.BlockSpec((blk, k), lambda i: (i, 0)),
            pl.BlockSpec((k, n), lambda i: (0, 0)),
        ],
        out_specs=pl.BlockSpec((blk, n), lambda i: (i, 0)),
        out_shape=jax.ShapeDtypeStruct((m, n), jnp.float32),
    )(x, w)


def _relu_mm_body(p_ref, w_ref, o_ref):
    h = jnp.maximum(p_ref[0] + p_ref[1], 0.0)
    o_ref[...] = jnp.dot(h, w_ref[...], preferred_element_type=jnp.float32)


def _relu_matmul(p, w):
    _, m, k = p.shape
    n = w.shape[1]
    blk = 1000
    return pl.pallas_call(
        _relu_mm_body,
        grid=(m // blk,),
        in_specs=[
            pl.BlockSpec((2, blk, k), lambda i: (0, i, 0)),
            pl.BlockSpec((k, n), lambda i: (0, 0)),
        ],
        out_specs=pl.BlockSpec((blk, n), lambda i: (i, 0)),
        out_shape=jax.ShapeDtypeStruct((m, n), jnp.float32),
    )(p, w)


def _final_body(p_ref, o_ref):
    o_ref[...] = (p_ref[0] + p_ref[1])[:, :OUT_DIM]


def _final_combine(p):
    _, m, k = p.shape
    blk = 1000
    return pl.pallas_call(
        _final_body,
        grid=(m // blk,),
        in_specs=[pl.BlockSpec((2, blk, k), lambda i: (0, i, 0))],
        out_specs=pl.BlockSpec((blk, OUT_DIM), lambda i: (i, 0)),
        out_shape=jax.ShapeDtypeStruct((m, OUT_DIM), jnp.float32),
    )(p)


def _make_edge_agg(d):
    mesh = plsc.VectorSubcoreMesh(core_axis_name="c", subcore_axis_name="s")

    @functools.partial(
        pl.kernel,
        out_type=jax.ShapeDtypeStruct((NC, N_NODES, d), jnp.float32),
        mesh=mesh,
        compiler_params=pltpu.CompilerParams(use_tc_tiling_on_sc=False),
        scratch_types=[
            pltpu.VMEM((CHUNK,), jnp.int32),
            pltpu.VMEM((CHUNK,), jnp.int32),
            pltpu.VMEM((CHUNK, d), jnp.float32),
            pltpu.VMEM_SHARED((N_NODES, d), jnp.float32),
            pltpu.SemaphoreType.DMA,
        ],
    )
    def agg(table_hbm, src_hbm, dst_hbm, zeros_hbm, out_hbm,
            src_v, dst_v, msg_v, acc_s, sem):
        cid = lax.axis_index("c")
        sid = lax.axis_index("s")
        row0 = sid * ROWS_PER_TILE

        pltpu.sync_copy(zeros_hbm.at[pl.ds(row0, ROWS_PER_TILE)],
                        acc_s.at[pl.ds(row0, ROWS_PER_TILE)])

        @pl.when(sid == NS - 1)
        def _zero_tail():
            pltpu.sync_copy(zeros_hbm.at[pl.ds(NS * ROWS_PER_TILE, ROWS_TAIL)],
                            acc_s.at[pl.ds(NS * ROWS_PER_TILE, ROWS_TAIL)])

        plsc.subcore_barrier()

        base0 = cid * (NS * EDGES_PER_TILE) + sid * EDGES_PER_TILE

        def body(i, carry):
            base = base0 + i * CHUNK
            pltpu.sync_copy(src_hbm.at[pl.ds(base, CHUNK)], src_v)
            pltpu.sync_copy(dst_hbm.at[pl.ds(base, CHUNK)], dst_v)
            pltpu.async_copy(table_hbm.at[src_v], msg_v, sem).wait()
            pltpu.sync_copy(msg_v, acc_s.at[dst_v], add=True)
            return carry

        lax.fori_loop(0, ITERS, body, 0)
        plsc.subcore_barrier()

        pltpu.sync_copy(acc_s.at[pl.ds(row0, ROWS_PER_TILE)],
                        out_hbm.at[cid, pl.ds(row0, ROWS_PER_TILE)])

        @pl.when(sid == NS - 1)
        def _drain_tail():
            pltpu.sync_copy(acc_s.at[pl.ds(NS * ROWS_PER_TILE, ROWS_TAIL)],
                            out_hbm.at[cid, pl.ds(NS * ROWS_PER_TILE, ROWS_TAIL)])

    return agg


_agg_d1 = _make_edge_agg(D1)
_agg_d2 = _make_edge_agg(D2)


def kernel(x, edge_index, W0, W1):
    src = edge_index[0].astype(jnp.int32)
    dst = edge_index[1].astype(jnp.int32)
    w0p = jnp.pad(W0, ((0, 0), (0, D1 - HIDDEN)))
    w1p = jnp.pad(W1, ((0, D1 - HIDDEN), (0, D2 - OUT_DIM)))
    z1 = jnp.zeros((N_NODES, D1), jnp.float32)
    z2 = jnp.zeros((N_NODES, D2), jnp.float32)

    hw = _matmul(x, w0p)
    p1 = _agg_d1(hw, src, dst, z1)
    h1w = _relu_matmul(p1, w1p)
    p2 = _agg_d2(h1w, src, dst, z2)
    return _final_combine(p2)


# speedup vs baseline: 7.2428x; 1.2846x over previous
import functools

import jax
import jax.numpy as jnp
from jax import lax
from jax.experimental import pallas as pl
from jax.experimental.pallas import tpu as pltpu
from jax.experimental.pallas import tpu_sc as plsc

N_NODES = 10000
N_EDGES = 320000
D_FEAT = 128
HIDDEN = 100
OUT_DIM = 10

D1 = 112
D2 = 16
NC = 2
NS = 16
CHUNK = 40
EDGES_PER_TILE = N_EDGES // (NC * NS)
ITERS = EDGES_PER_TILE // CHUNK
ROWS_PER_TILE = 624
ROWS_TAIL = N_NODES - NS * ROWS_PER_TILE


def _mm_body(x_ref, w_ref, o_ref):
    o_ref[...] = jnp.dot(x_ref[...], w_ref[...],
                         preferred_element_type=jnp.float32)


def _matmul(x, w):
    m, k = x.shape
    n = w.shape[1]
    blk = 1000
    return pl.pallas_call(
        _mm_body,
        grid=(m // blk,),
        in_specs=[
            pl.BlockSpec((blk, k), lambda i: (i, 0)),
            pl.BlockSpec((k, n), lambda i: (0, 0)),
        ],
        out_specs=pl.BlockSpec((blk, n), lambda i: (i, 0)),
        out_shape=jax.ShapeDtypeStruct((m, n), jnp.float32),
    )(x, w)


def _relu_mm_body(p_ref, w_ref, o_ref):
    h = jnp.maximum(p_ref[0] + p_ref[1], 0.0)
    o_ref[...] = jnp.dot(h, w_ref[...], preferred_element_type=jnp.float32)


def _relu_matmul(p, w):
    _, m, k = p.shape
    n = w.shape[1]
    blk = 1000
    return pl.pallas_call(
        _relu_mm_body,
        grid=(m // blk,),
        in_specs=[
            pl.BlockSpec((2, blk, k), lambda i: (0, i, 0)),
            pl.BlockSpec((k, n), lambda i: (0, 0)),
        ],
        out_specs=pl.BlockSpec((blk, n), lambda i: (i, 0)),
        out_shape=jax.ShapeDtypeStruct((m, n), jnp.float32),
    )(p, w)


def _final_body(p_ref, o_ref):
    o_ref[...] = (p_ref[0] + p_ref[1])[:, :OUT_DIM]


def _final_combine(p):
    _, m, k = p.shape
    blk = 1000
    return pl.pallas_call(
        _final_body,
        grid=(m // blk,),
        in_specs=[pl.BlockSpec((2, blk, k), lambda i: (0, i, 0))],
        out_specs=pl.BlockSpec((blk, OUT_DIM), lambda i: (i, 0)),
        out_shape=jax.ShapeDtypeStruct((m, OUT_DIM), jnp.float32),
    )(p)


def _make_edge_agg(d):
    mesh = plsc.VectorSubcoreMesh(core_axis_name="c", subcore_axis_name="s")

    @functools.partial(
        pl.kernel,
        out_type=jax.ShapeDtypeStruct((NC, N_NODES, d), jnp.float32),
        mesh=mesh,
        compiler_params=pltpu.CompilerParams(use_tc_tiling_on_sc=False),
        scratch_types=[
            pltpu.VMEM((ITERS, CHUNK), jnp.int32),
            pltpu.VMEM((ITERS, CHUNK), jnp.int32),
            [pltpu.VMEM((CHUNK, d), jnp.float32) for _ in range(2)],
            pltpu.VMEM_SHARED((N_NODES, d), jnp.float32),
            pltpu.SemaphoreType.DMA,
        ],
    )
    def agg(table_hbm, src_hbm, dst_hbm, zeros_hbm, out_hbm,
            src_v, dst_v, msgs, acc_s, gsem):
        cid = lax.axis_index("c")
        sid = lax.axis_index("s")
        row0 = sid * ROWS_PER_TILE

        crow0 = (cid * NS + sid) * ITERS
        pltpu.async_copy(src_hbm.at[pl.ds(crow0, ITERS)], src_v, gsem)
        pltpu.async_copy(dst_hbm.at[pl.ds(crow0, ITERS)], dst_v, gsem)

        pltpu.sync_copy(zeros_hbm.at[pl.ds(row0, ROWS_PER_TILE)],
                        acc_s.at[pl.ds(row0, ROWS_PER_TILE)])

        @pl.when(sid == NS - 1)
        def _zero_tail():
            pltpu.sync_copy(zeros_hbm.at[pl.ds(NS * ROWS_PER_TILE, ROWS_TAIL)],
                            acc_s.at[pl.ds(NS * ROWS_PER_TILE, ROWS_TAIL)])

        pltpu.make_async_copy(src_hbm.at[pl.ds(crow0, ITERS)], src_v,
                              gsem).wait()
        pltpu.make_async_copy(dst_hbm.at[pl.ds(crow0, ITERS)], dst_v,
                              gsem).wait()
        plsc.subcore_barrier()

        def fire_gather(i, b):
            pltpu.async_copy(table_hbm.at[src_v.at[i]], msgs[b], gsem)

        def wait_gather(i, b):
            pltpu.make_async_copy(table_hbm.at[src_v.at[i]], msgs[b],
                                  gsem).wait()

        fire_gather(0, 0)

        def outer(k, carry):
            for p in range(2):
                i = 2 * k + p
                wait_gather(i, p)

                @pl.when(i < ITERS - 1)
                def _prefetch():
                    fire_gather(i + 1, 1 - p)

                pltpu.sync_copy(msgs[p], acc_s.at[dst_v.at[i]], add=True)
            return carry

        lax.fori_loop(0, ITERS // 2, outer, 0)
        plsc.subcore_barrier()

        pltpu.sync_copy(acc_s.at[pl.ds(row0, ROWS_PER_TILE)],
                        out_hbm.at[cid, pl.ds(row0, ROWS_PER_TILE)])

        @pl.when(sid == NS - 1)
        def _drain_tail():
            pltpu.sync_copy(acc_s.at[pl.ds(NS * ROWS_PER_TILE, ROWS_TAIL)],
                            out_hbm.at[cid, pl.ds(NS * ROWS_PER_TILE, ROWS_TAIL)])

    return agg


_agg_d1 = _make_edge_agg(D1)
_agg_d2 = _make_edge_agg(D2)


def kernel(x, edge_index, W0, W1):
    src = edge_index[0].astype(jnp.int32).reshape(N_EDGES // CHUNK, CHUNK)
    dst = edge_index[1].astype(jnp.int32).reshape(N_EDGES // CHUNK, CHUNK)
    w0p = jnp.pad(W0, ((0, 0), (0, D1 - HIDDEN)))
    w1p = jnp.pad(W1, ((0, D1 - HIDDEN), (0, D2 - OUT_DIM)))
    z1 = jnp.zeros((N_NODES, D1), jnp.float32)
    z2 = jnp.zeros((N_NODES, D2), jnp.float32)

    hw = _matmul(x, w0p)
    p1 = _agg_d1(hw, src, dst, z1)
    h1w = _relu_matmul(p1, w1p)
    p2 = _agg_d2(h1w, src, dst, z2)
    return _final_combine(p2)
